# baseline (speedup 1.0000x reference)
import jax
import jax.numpy as jnp
from jax import lax
from jax.experimental import pallas as pl
from jax.experimental.pallas import tpu as pltpu
from jax.experimental.pallas import tpu_sc as plsc

N_TAB = 1000
BATCH = 16384
LANES = 16
NUM_WORKERS = 32
B_PER_W = BATCH // NUM_WORKERS
CHUNK = 128
N_CHUNKS = B_PER_W // CHUNK
ITERS = B_PER_W // LANES
GRP = CHUNK // LANES


def _sc_body(ii_hbm, wa_hbm, wb_hbm, wc_hbm, wab_hbm, wcb_hbm,
             out_hbm,
             idx3_v, iab_v, icb_v, pab_v, pcb_v,
             t1_v, t2_v, wa_v, wb_v, wc_v, out_v, sem_i, sem_t, sem_s):
    nc = 2
    wid = lax.axis_index("s") * nc + lax.axis_index("c")
    base = wid * B_PER_W

    hi = pltpu.async_copy(ii_hbm.at[pl.ds(base * 3, B_PER_W * 3)], idx3_v,
                          sem_i)
    ht = [pltpu.async_copy(wa_hbm, wa_v, sem_t),
          pltpu.async_copy(wb_hbm, wb_v, sem_t),
          pltpu.async_copy(wc_hbm, wc_v, sem_t)]
    hi.wait()

    lane3 = jnp.arange(LANES, dtype=jnp.int32) * 3

    for j in range(N_CHUNKS):
        @pl.loop(0, GRP)
        def _pass0(g, j=j):
            o3 = (j * CHUNK + g * LANES) * 3
            a = plsc.load_gather(idx3_v, [lane3 + o3])
            b = plsc.load_gather(idx3_v, [lane3 + (o3 + 1)])
            c = plsc.load_gather(idx3_v, [lane3 + (o3 + 2)])
            iab_v[j, pl.ds(g * LANES, LANES)] = a * N_TAB + b
            icb_v[j, pl.ds(g * LANES, LANES)] = c * N_TAB + a

    hs = []
    for j in range(N_CHUNKS):
        hs.append(pltpu.async_copy(wab_hbm.at[iab_v.at[j]], pab_v.at[j], sem_s))
        hs.append(pltpu.async_copy(wcb_hbm.at[icb_v.at[j]], pcb_v.at[j], sem_s))

    for h in ht:
        h.wait()

    @pl.loop(0, ITERS)
    def _pass1(i):
        o = i * LANES
        o3 = o * 3
        a = plsc.load_gather(idx3_v, [lane3 + o3])
        b = plsc.load_gather(idx3_v, [lane3 + (o3 + 1)])
        c = plsc.load_gather(idx3_v, [lane3 + (o3 + 2)])
        pa = plsc.load_gather(wa_v, [a])
        pb = plsc.load_gather(wb_v, [b])
        pc = plsc.load_gather(wc_v, [c])
        t1_v[pl.ds(o, LANES)] = pa + pc
        t2_v[pl.ds(o, LANES)] = pb / (pa * pc)

    for h in hs:
        h.wait()

    for j in range(N_CHUNKS):
        @pl.loop(0, GRP)
        def _pass2(g, j=j):
            o = j * CHUNK + g * LANES
            jo = g * LANES
            pab = pab_v[j, pl.ds(jo, LANES)]
            pcb = pcb_v[j, pl.ds(jo, LANES)]
            out_v[pl.ds(o, LANES)] = (t1_v[pl.ds(o, LANES)]
                                      + pab * pcb * t2_v[pl.ds(o, LANES)])
    pltpu.sync_copy(out_v, out_hbm.at[pl.ds(base, B_PER_W)])


@jax.jit
def _run(ii, w_A, w_B, w_C, wab_flat, wcb_flat):
    mesh = plsc.VectorSubcoreMesh(core_axis_name="c", subcore_axis_name="s")
    f = pl.kernel(
        _sc_body,
        mesh=mesh,
        compiler_params=pltpu.CompilerParams(needs_layout_passes=False),
        out_type=jax.ShapeDtypeStruct((BATCH,), jnp.float32),
        scratch_types=[
            pltpu.VMEM((B_PER_W * 3,), jnp.int32),
            pltpu.VMEM((N_CHUNKS, CHUNK), jnp.int32),
            pltpu.VMEM((N_CHUNKS, CHUNK), jnp.int32),
            pltpu.VMEM((N_CHUNKS, CHUNK), jnp.float32),
            pltpu.VMEM((N_CHUNKS, CHUNK), jnp.float32),
            pltpu.VMEM((B_PER_W,), jnp.float32),
            pltpu.VMEM((B_PER_W,), jnp.float32),
            pltpu.VMEM((N_TAB,), jnp.float32),
            pltpu.VMEM((N_TAB,), jnp.float32),
            pltpu.VMEM((N_TAB,), jnp.float32),
            pltpu.VMEM((B_PER_W,), jnp.float32),
            pltpu.SemaphoreType.DMA,
            pltpu.SemaphoreType.DMA,
            pltpu.SemaphoreType.DMA,
        ],
    )
    return f(ii, w_A, w_B, w_C, wab_flat, wcb_flat)


def kernel(inputs, w_A, w_B, w_C, w_AB, w_CB):
    ii = inputs.reshape(-1)
    wab_flat = w_AB.reshape(-1)
    wcb_flat = w_CB.reshape(-1)
    return _run(ii, w_A, w_B, w_C, wab_flat, wcb_flat)


# speedup vs baseline: 1.0950x; 1.0950x over previous
import jax
import jax.numpy as jnp
from jax import lax
from jax.experimental import pallas as pl
from jax.experimental.pallas import tpu as pltpu
from jax.experimental.pallas import tpu_sc as plsc

N_TAB = 1000
BATCH = 16384
LANES = 16
NUM_WORKERS = 32
B_PER_W = BATCH // NUM_WORKERS
CHUNK = 128
N_CHUNKS = B_PER_W // CHUNK
ITERS = B_PER_W // LANES
GRP = CHUNK // LANES


def _sc_body(in_hbm, wa_hbm, wb_hbm, wc_hbm, wab_hbm, wcb_hbm,
             out_hbm,
             idx3_v, iab_v, icb_v, pab_v, pcb_v,
             t1_v, t2_v, wa_v, wb_v, wc_v, out_v, sem_i, sem_t, sem_s):
    nc = 2
    wid = lax.axis_index("s") * nc + lax.axis_index("c")
    base = wid * B_PER_W

    hi = pltpu.async_copy(in_hbm.at[pl.ds(base, B_PER_W)], idx3_v, sem_i)
    ht = [pltpu.async_copy(wa_hbm, wa_v, sem_t),
          pltpu.async_copy(wb_hbm, wb_v, sem_t),
          pltpu.async_copy(wc_hbm, wc_v, sem_t)]
    hi.wait()

    lane = jnp.arange(LANES, dtype=jnp.int32)

    def _abc(o):
        rows = lane + o
        a = plsc.load_gather(idx3_v, [rows, jnp.zeros((LANES,), jnp.int32)])
        b = plsc.load_gather(idx3_v, [rows, jnp.ones((LANES,), jnp.int32)])
        c = plsc.load_gather(idx3_v, [rows, jnp.full((LANES,), 2, jnp.int32)])
        return a, b, c

    for j in range(N_CHUNKS):
        @pl.loop(0, GRP)
        def _pass0(g, j=j):
            a, b, c = _abc(j * CHUNK + g * LANES)
            iab_v[j, pl.ds(g * LANES, LANES)] = a * N_TAB + b
            icb_v[j, pl.ds(g * LANES, LANES)] = c * N_TAB + a

    hs = []
    for j in range(N_CHUNKS):
        hs.append(pltpu.async_copy(wab_hbm.at[iab_v.at[j]], pab_v.at[j], sem_s))
        hs.append(pltpu.async_copy(wcb_hbm.at[icb_v.at[j]], pcb_v.at[j], sem_s))

    for h in ht:
        h.wait()

    @pl.loop(0, ITERS)
    def _pass1(i):
        o = i * LANES
        a, b, c = _abc(o)
        pa = plsc.load_gather(wa_v, [a])
        pb = plsc.load_gather(wb_v, [b])
        pc = plsc.load_gather(wc_v, [c])
        t1_v[pl.ds(o, LANES)] = pa + pc
        t2_v[pl.ds(o, LANES)] = pb / (pa * pc)

    for h in hs:
        h.wait()

    for j in range(N_CHUNKS):
        @pl.loop(0, GRP)
        def _pass2(g, j=j):
            o = j * CHUNK + g * LANES
            jo = g * LANES
            pab = pab_v[j, pl.ds(jo, LANES)]
            pcb = pcb_v[j, pl.ds(jo, LANES)]
            out_v[pl.ds(o, LANES)] = (t1_v[pl.ds(o, LANES)]
                                      + pab * pcb * t2_v[pl.ds(o, LANES)])
    pltpu.sync_copy(out_v, out_hbm.at[pl.ds(base, B_PER_W)])


@jax.jit
def _run(inputs, w_A, w_B, w_C, wab_flat, wcb_flat):
    mesh = plsc.VectorSubcoreMesh(core_axis_name="c", subcore_axis_name="s")
    f = pl.kernel(
        _sc_body,
        mesh=mesh,
        compiler_params=pltpu.CompilerParams(needs_layout_passes=False),
        out_type=jax.ShapeDtypeStruct((BATCH,), jnp.float32),
        scratch_types=[
            pltpu.VMEM((B_PER_W, 3), jnp.int32),
            pltpu.VMEM((N_CHUNKS, CHUNK), jnp.int32),
            pltpu.VMEM((N_CHUNKS, CHUNK), jnp.int32),
            pltpu.VMEM((N_CHUNKS, CHUNK), jnp.float32),
            pltpu.VMEM((N_CHUNKS, CHUNK), jnp.float32),
            pltpu.VMEM((B_PER_W,), jnp.float32),
            pltpu.VMEM((B_PER_W,), jnp.float32),
            pltpu.VMEM((N_TAB,), jnp.float32),
            pltpu.VMEM((N_TAB,), jnp.float32),
            pltpu.VMEM((N_TAB,), jnp.float32),
            pltpu.VMEM((B_PER_W,), jnp.float32),
            pltpu.SemaphoreType.DMA,
            pltpu.SemaphoreType.DMA,
            pltpu.SemaphoreType.DMA,
        ],
    )
    return f(inputs, w_A, w_B, w_C, wab_flat, wcb_flat)


def kernel(inputs, w_A, w_B, w_C, w_AB, w_CB):
    wab_flat = w_AB.reshape(-1)
    wcb_flat = w_CB.reshape(-1)
    return _run(inputs, w_A, w_B, w_C, wab_flat, wcb_flat)


# speedup vs baseline: 1.3697x; 1.2509x over previous
import jax
import jax.numpy as jnp
from jax import lax
from jax.experimental import pallas as pl
from jax.experimental.pallas import tpu as pltpu
from jax.experimental.pallas import tpu_sc as plsc

N_TAB = 1000
BATCH = 16384
LANES = 16
NUM_WORKERS = 32
B_PER_W = BATCH // NUM_WORKERS
CHUNK = 128
N_CHUNKS = B_PER_W // CHUNK
ITERS = B_PER_W // LANES
GRP = CHUNK // LANES


def _sc_body(ia_hbm, ib_hbm, ic_hbm, wa_hbm, wb_hbm, wc_hbm, wab_hbm, wcb_hbm,
             out_hbm,
             idxa_v, idxb_v, idxc_v, iab_v, icb_v, pab_v, pcb_v,
             t1_v, t2_v, wa_v, wb_v, wc_v, out_v, sem_i, sem_t,
             sem_s0, sem_s1, sem_s2, sem_s3):
    nc = 2
    wid = lax.axis_index("s") * nc + lax.axis_index("c")
    base = wid * B_PER_W

    hi = [pltpu.async_copy(ia_hbm.at[pl.ds(base, B_PER_W)], idxa_v, sem_i),
          pltpu.async_copy(ib_hbm.at[pl.ds(base, B_PER_W)], idxb_v, sem_i),
          pltpu.async_copy(ic_hbm.at[pl.ds(base, B_PER_W)], idxc_v, sem_i)]
    ht = [pltpu.async_copy(wa_hbm, wa_v, sem_t),
          pltpu.async_copy(wb_hbm, wb_v, sem_t),
          pltpu.async_copy(wc_hbm, wc_v, sem_t)]
    for h in hi:
        h.wait()

    sem_c = [sem_s0, sem_s1, sem_s2, sem_s3]
    hs = []
    for j in range(N_CHUNKS):
        @pl.loop(0, GRP, unroll=2)
        def _pass0(g, j=j):
            o = j * CHUNK + g * LANES
            a = idxa_v[pl.ds(o, LANES)]
            b = idxb_v[pl.ds(o, LANES)]
            c = idxc_v[pl.ds(o, LANES)]
            iab_v[j, pl.ds(g * LANES, LANES)] = a * N_TAB + b
            icb_v[j, pl.ds(g * LANES, LANES)] = c * N_TAB + a
        hs.append((
            pltpu.async_copy(wab_hbm.at[iab_v.at[j]], pab_v.at[j], sem_c[j]),
            pltpu.async_copy(wcb_hbm.at[icb_v.at[j]], pcb_v.at[j], sem_c[j]),
        ))

    for h in ht:
        h.wait()

    @pl.loop(0, ITERS, unroll=2)
    def _pass1(i):
        o = i * LANES
        a = idxa_v[pl.ds(o, LANES)]
        b = idxb_v[pl.ds(o, LANES)]
        c = idxc_v[pl.ds(o, LANES)]
        pa = plsc.load_gather(wa_v, [a])
        pb = plsc.load_gather(wb_v, [b])
        pc = plsc.load_gather(wc_v, [c])
        t1_v[pl.ds(o, LANES)] = pa + pc
        t2_v[pl.ds(o, LANES)] = pb / (pa * pc)

    for j in range(N_CHUNKS):
        hs[j][0].wait()
        hs[j][1].wait()

        @pl.loop(0, GRP, unroll=2)
        def _pass2(g, j=j):
            o = j * CHUNK + g * LANES
            jo = g * LANES
            pab = pab_v[j, pl.ds(jo, LANES)]
            pcb = pcb_v[j, pl.ds(jo, LANES)]
            out_v[pl.ds(o, LANES)] = (t1_v[pl.ds(o, LANES)]
                                      + pab * pcb * t2_v[pl.ds(o, LANES)])
    pltpu.sync_copy(out_v, out_hbm.at[pl.ds(base, B_PER_W)])


@jax.jit
def _run(ia, ib, ic, w_A, w_B, w_C, wab_flat, wcb_flat):
    mesh = plsc.VectorSubcoreMesh(core_axis_name="c", subcore_axis_name="s")
    f = pl.kernel(
        _sc_body,
        mesh=mesh,
        compiler_params=pltpu.CompilerParams(needs_layout_passes=False),
        out_type=jax.ShapeDtypeStruct((BATCH,), jnp.float32),
        scratch_types=[
            pltpu.VMEM((B_PER_W,), jnp.int32),
            pltpu.VMEM((B_PER_W,), jnp.int32),
            pltpu.VMEM((B_PER_W,), jnp.int32),
            pltpu.VMEM((N_CHUNKS, CHUNK), jnp.int32),
            pltpu.VMEM((N_CHUNKS, CHUNK), jnp.int32),
            pltpu.VMEM((N_CHUNKS, CHUNK), jnp.float32),
            pltpu.VMEM((N_CHUNKS, CHUNK), jnp.float32),
            pltpu.VMEM((B_PER_W,), jnp.float32),
            pltpu.VMEM((B_PER_W,), jnp.float32),
            pltpu.VMEM((N_TAB,), jnp.float32),
            pltpu.VMEM((N_TAB,), jnp.float32),
            pltpu.VMEM((N_TAB,), jnp.float32),
            pltpu.VMEM((B_PER_W,), jnp.float32),
            pltpu.SemaphoreType.DMA,
            pltpu.SemaphoreType.DMA,
            pltpu.SemaphoreType.DMA,
            pltpu.SemaphoreType.DMA,
            pltpu.SemaphoreType.DMA,
            pltpu.SemaphoreType.DMA,
        ],
    )
    return f(ia, ib, ic, w_A, w_B, w_C, wab_flat, wcb_flat)


def kernel(inputs, w_A, w_B, w_C, w_AB, w_CB):
    ia = inputs[:, 0]
    ib = inputs[:, 1]
    ic = inputs[:, 2]
    wab_flat = w_AB.reshape(-1)
    wcb_flat = w_CB.reshape(-1)
    return _run(ia, ib, ic, w_A, w_B, w_C, wab_flat, wcb_flat)
